# Initial kernel scaffold; baseline (speedup 1.0000x reference)
import jax
import jax.numpy as jnp
from jax.experimental import pallas as pl


def kernel(input_ids, logits, input_vector, random_vectors):
    raise NotImplementedError("write your pallas kernel here")


# speedup vs baseline: 2.7063x; 2.7063x over previous
import functools

import jax
import jax.numpy as jnp
from jax import lax
from jax.experimental import pallas as pl
from jax.experimental.pallas import tpu as pltpu
from jax.experimental.pallas import tpu_sc as plsc

B = 16
VOCAB = 1000000
EMBED = 768
BBITS = 16
SEED = 42
TOP_P = 0.9

HALF = VOCAB // 2
CHUNK = 10000
NCHUNK = HALF // CHUNK
NVEC = CHUNK // 16
NB1 = 4096
NB2 = 4096
NB3 = 256
HWORDS = NB1 * 16
TIE_CAP = 64
TIE_WORDS = TIE_CAP + 16
FILL = -100000.0
MARK = 100000.0
ROWS_PER_SC = 8
MERGE_CHUNK = 10000
MERGE_STRIDE = 10112
REC_STRIDE = 128

_LANE = None


def _keys_of(x):
    u = plsc.bitcast(x, jnp.uint32)
    s = u >> jnp.uint32(31)
    flip = (jnp.uint32(0) - s) | jnp.uint32(0x80000000)
    return u ^ flip


def _sc_body(logits_hbm, nlx_hbm, out_hbm,
             xbuf, nbuf, hist, tieidx, tienlx, recfv, reciv, patch,
             hist_sh, recf_sh, reci_sh, tidx_sh, tnlx_sh):
    c = lax.axis_index("c")
    s = lax.axis_index("s")
    row_l = s // 2
    half = s % 2
    row = c * ROWS_PER_SC + row_l
    base = half * HALF
    rbase = row * VOCAB + base

    lane = jnp.arange(16, dtype=jnp.int32)
    zero16 = jnp.zeros((16,), jnp.float32)
    inf16 = jnp.full((16,), jnp.inf, dtype=jnp.float32)

    def exf(vec, n):
        return jnp.sum(jnp.where(lane == n, vec, 0.0))

    def exi(vec, n):
        return jnp.sum(jnp.where(lane == n, vec, 0))

    def zero_hist():
        def bd(i, carry):
            hist[pl.ds(i * 16, 16)] = zero16
            return carry
        lax.fori_loop(0, HWORDS // 16, bd, 0)

    def hist_pass(shift, nbits, prefix_shift, prefix):
        bmask = jnp.uint32((1 << nbits) - 1)
        shift_u = jnp.uint32(shift)

        def chunk_body(ci, carry):
            pltpu.sync_copy(
                logits_hbm.at[pl.ds(rbase + ci * CHUNK, CHUNK)], xbuf)

            def vec_body(vi, c2):
                x = xbuf[pl.ds(vi * 16, 16)]
                w = jnp.exp(x)
                k = _keys_of(x)
                b = ((k >> shift_u) & bmask).astype(jnp.int32)
                addr = b * 16 + lane
                if prefix_shift is None:
                    plsc.addupdate_scatter(hist, [addr], w)
                else:
                    pred = (k >> jnp.uint32(prefix_shift)) == prefix
                    plsc.addupdate_scatter(hist, [addr], w, mask=pred)
                return c2
            lax.fori_loop(0, NVEC, vec_body, 0)
            return carry
        lax.fori_loop(0, NCHUNK, chunk_body, 0)

    def merge_hist():
        slot = (row_l * 2 + half) * MERGE_STRIDE
        oslot = (row_l * 2 + (1 - half)) * MERGE_STRIDE
        total = zero16
        for off, ln in ((0, 10000), (10000, 10000), (20000, 10000),
                        (30000, 10000), (40000, 10000), (50000, 10000),
                        (60000, 5536)):
            pltpu.sync_copy(hist.at[pl.ds(off, ln)],
                            hist_sh.at[pl.ds(slot, ln)])
            plsc.subcore_barrier()
            pltpu.sync_copy(hist_sh.at[pl.ds(oslot, ln)],
                            nbuf.at[pl.ds(0, ln)])

            def vb(vi, tot):
                cur = hist[pl.ds(off + vi * 16, 16)]
                oth = nbuf[pl.ds(vi * 16, 16)]
                m = cur + oth
                hist[pl.ds(off + vi * 16, 16)] = m
                return tot + m
            total = lax.fori_loop(0, ln // 16, vb, total)
            plsc.subcore_barrier()
        return jnp.sum(total)

    def scan_level(nbuckets, thresh, base_mass):
        def cond(st):
            j, acc, found, bst, m0, ln_b, ln_m0 = st
            return jnp.logical_and(j >= 0, jnp.logical_not(found))

        def body(st):
            j, acc, found, bst, m0, ln_b, ln_m0 = st
            bm = jnp.sum(hist[pl.ds(j * 16, 16)])
            acc2 = acc + bm
            crossed = acc2 >= thresh
            nonempty = bm > 0.0
            return (j - 1, acc2, crossed,
                    jnp.where(crossed, j, bst),
                    jnp.where(crossed, acc, m0),
                    jnp.where(nonempty, j, ln_b),
                    jnp.where(nonempty, acc, ln_m0))

        init = (jnp.int32(nbuckets - 1), base_mass, False,
                jnp.int32(0), base_mass, jnp.int32(0), base_mass)
        j, acc, found, bst, m0, ln_b, ln_m0 = lax.while_loop(
            cond, body, init)
        bst = jnp.where(found, bst, ln_b)
        m0 = jnp.where(found, m0, ln_m0)
        return bst, m0

    zero_hist()
    hist_pass(20, 12, None, None)
    e_total = merge_hist()
    thresh = jnp.float32(TOP_P) * e_total
    b1, m0_1 = scan_level(NB1, thresh, jnp.float32(0.0))
    b1u = b1.astype(jnp.uint32)

    zero_hist()
    hist_pass(8, 12, 20, b1u)
    merge_hist()
    b2, m0_2 = scan_level(NB2, thresh, m0_1)
    b2u = b2.astype(jnp.uint32)

    zero_hist()
    pref24 = (b1u << jnp.uint32(12)) | b2u
    hist_pass(0, 8, 8, pref24)
    merge_hist()
    b3, m0_3 = scan_level(NB3, thresh, m0_2)
    b3u = b3.astype(jnp.uint32)

    kstar = (b1u << jnp.uint32(20)) | (b2u << jnp.uint32(8)) | b3u
    m0 = m0_3

    kv = jnp.broadcast_to(kstar, (16,))
    sbit = kv >> jnp.uint32(31)
    uv = jnp.where(sbit == jnp.uint32(1),
                   kv ^ jnp.uint32(0x80000000), ~kv)
    wv_vec = jnp.exp(plsc.bitcast(uv, jnp.float32))
    w_v = jnp.max(wv_vec)

    qr_vec = jnp.broadcast_to(thresh - m0, (16,)) / wv_vec
    qr = jnp.minimum(jnp.max(qr_vec), jnp.float32(1.5e9))
    r0 = qr.astype(jnp.int32)
    bump = (r0.astype(jnp.float32) * w_v + m0 < thresh).astype(jnp.int32)
    r = jnp.maximum(r0 + bump, 1)

    def sel_chunk(ci, st):
        best, bidx, cnt, manl, maix = st
        pltpu.sync_copy(
            logits_hbm.at[pl.ds(rbase + ci * CHUNK, CHUNK)], xbuf)
        pltpu.sync_copy(
            nlx_hbm.at[pl.ds(rbase + ci * CHUNK, CHUNK)], nbuf)

        def vec_body(vi, st2):
            best, bidx, cnt, manl, maix = st2
            x = xbuf[pl.ds(vi * 16, 16)]
            nl = nbuf[pl.ds(vi * 16, 16)]
            w = jnp.exp(x)
            k = _keys_of(x)
            idxv = base + ci * CHUNK + vi * 16 + lane
            certain = k > kstar
            sc = jnp.where(certain, nl / w, jnp.inf)
            better = sc < best
            best = jnp.where(better, sc, best)
            bidx = jnp.where(better, idxv, bidx)
            tiem = k == kstar
            offs = jnp.minimum(jnp.max(cnt), TIE_CAP)
            plsc.store_compressed(tieidx.at[pl.ds(offs, 16)], idxv,
                                  mask=tiem)
            tnl = jnp.where(tiem, nl, jnp.inf)
            plsc.store_compressed(tienlx.at[pl.ds(offs, 16)], tnl,
                                  mask=tiem)
            cnt = cnt + plsc.all_reduce_population_count(tiem)
            tbet = tnl < manl
            manl = jnp.where(tbet, tnl, manl)
            maix = jnp.where(tbet, idxv, maix)
            return best, bidx, cnt, manl, maix

        return lax.fori_loop(0, NVEC, vec_body,
                             (best, bidx, cnt, manl, maix))

    izero = jnp.zeros((16,), jnp.int32)
    best, bidx, cnt, manl, maix = lax.fori_loop(
        0, NCHUNK, sel_chunk, (inf16, izero, izero, inf16, izero))

    bigi = jnp.int32(2 ** 30)
    bestm = jnp.min(best)
    besti = jnp.min(jnp.where(best == bestm, bidx, bigi))
    ncnt = jnp.max(cnt)
    manl_s = jnp.min(manl)
    maix_s = jnp.min(jnp.where(manl == manl_s, maix, bigi))

    recf = jnp.where(lane == 0, bestm, jnp.where(lane == 1, manl_s, 0.0))
    reci = jnp.where(lane == 0, besti,
                     jnp.where(lane == 1, ncnt,
                               jnp.where(lane == 2, maix_s, 0)))
    recfv[...] = recf
    reciv[...] = reci
    myslot = row_l * 2 + half
    pltpu.sync_copy(recfv, recf_sh.at[pl.ds(myslot * REC_STRIDE, 16)])
    pltpu.sync_copy(reciv, reci_sh.at[pl.ds(myslot * REC_STRIDE, 16)])
    pltpu.sync_copy(tieidx, tidx_sh.at[pl.ds(myslot * REC_STRIDE, TIE_WORDS)])
    pltpu.sync_copy(tienlx, tnlx_sh.at[pl.ds(myslot * REC_STRIDE, TIE_WORDS)])
    plsc.subcore_barrier()

    s0 = row_l * 2
    pltpu.sync_copy(recf_sh.at[pl.ds(s0 * REC_STRIDE, 16)], recfv)
    pltpu.sync_copy(reci_sh.at[pl.ds(s0 * REC_STRIDE, 16)], reciv)
    rf0 = recfv[...]
    ri0 = reciv[...]
    bm0 = exf(rf0, 0)
    manl0 = exf(rf0, 1)
    bi0 = exi(ri0, 0)
    n0 = exi(ri0, 1)
    mai0 = exi(ri0, 2)
    s1 = row_l * 2 + 1
    pltpu.sync_copy(recf_sh.at[pl.ds(s1 * REC_STRIDE, 16)], recfv)
    pltpu.sync_copy(reci_sh.at[pl.ds(s1 * REC_STRIDE, 16)], reciv)
    rf1 = recfv[...]
    ri1 = reciv[...]
    bm1 = exf(rf1, 0)
    manl1 = exf(rf1, 1)
    bi1 = exi(ri1, 0)
    n1 = exi(ri1, 1)
    mai1 = exi(ri1, 2)

    k0 = jnp.minimum(r, n0)
    k1 = jnp.minimum(jnp.maximum(r - n0, 0), n1)
    use_all = jnp.logical_or(r >= n0 + n1,
                             jnp.logical_or(k0 > TIE_CAP, k1 > TIE_CAP))

    def walk(hh, kk, st):
        hs = (row_l * 2 + hh) * REC_STRIDE
        pltpu.sync_copy(tidx_sh.at[pl.ds(hs, TIE_WORDS)], tieidx)
        pltpu.sync_copy(tnlx_sh.at[pl.ds(hs, TIE_WORDS)], tienlx)

        def wb(j, st2):
            mn, mi = st2
            lm = (j * 16 + lane) < kk
            vals = jnp.where(lm, tienlx[pl.ds(j * 16, 16)], jnp.inf)
            ids = tieidx[pl.ds(j * 16, 16)]
            bet = vals < mn
            return jnp.where(bet, vals, mn), jnp.where(bet, ids, mi)
        return lax.fori_loop(0, TIE_CAP // 16, wb, st)

    tmn, tmi = walk(0, k0, (inf16, izero))
    tmn, tmi = walk(1, k1, (tmn, tmi))
    t_nl = jnp.min(tmn)
    t_ix = jnp.min(jnp.where(tmn == t_nl, tmi, bigi))

    all_nl = jnp.where(manl0 <= manl1, manl0, manl1)
    all_ix = jnp.where(manl0 <= manl1, mai0, mai1)
    tie_nl = jnp.where(use_all, all_nl, t_nl)
    tie_ix = jnp.where(use_all, all_ix, t_ix)
    tie_sc = jnp.max(jnp.broadcast_to(tie_nl, (16,)) / wv_vec)

    ws = bm0
    wi = bi0
    upd = bm1 < ws
    ws = jnp.where(upd, bm1, ws)
    wi = jnp.where(upd, bi1, wi)
    updt = tie_sc < ws
    wi = jnp.where(updt, tie_ix, wi)

    fill16 = jnp.full((16,), FILL, dtype=jnp.float32)

    def fb(i, carry):
        hist[pl.ds(i * 16, 16)] = fill16
        return carry
    lax.fori_loop(0, HWORDS // 16, fb, 0)
    for i in range(7):
        pltpu.sync_copy(hist,
                        out_hbm.at[pl.ds(rbase + i * HWORDS, HWORDS)])
    rem = HALF - 7 * HWORDS
    pltpu.sync_copy(hist.at[pl.ds(0, rem)],
                    out_hbm.at[pl.ds(rbase + 7 * HWORDS, rem)])

    own = jnp.logical_and(wi >= base, wi < base + HALF)

    @pl.when(own)
    def _patch():
        pb = wi & jnp.int32(-16)
        off = wi - pb
        patch[...] = jnp.where(lane == off, MARK, FILL)
        dst = pl.multiple_of(row * VOCAB + pb, 16)
        pltpu.sync_copy(patch, out_hbm.at[pl.ds(dst, 16)])


_mesh = plsc.VectorSubcoreMesh(core_axis_name="c", subcore_axis_name="s",
                               num_cores=2, num_subcores=16)

_sc_call = functools.partial(
    pl.kernel,
    out_type=jax.ShapeDtypeStruct((B * VOCAB,), jnp.float32),
    mesh=_mesh,
    compiler_params=pltpu.CompilerParams(needs_layout_passes=False),
    scratch_types=[
        pltpu.VMEM((CHUNK,), jnp.float32),
        pltpu.VMEM((CHUNK,), jnp.float32),
        pltpu.VMEM((HWORDS,), jnp.float32),
        pltpu.VMEM((TIE_WORDS,), jnp.int32),
        pltpu.VMEM((TIE_WORDS,), jnp.float32),
        pltpu.VMEM((16,), jnp.float32),
        pltpu.VMEM((16,), jnp.int32),
        pltpu.VMEM((16,), jnp.float32),
        pltpu.VMEM_SHARED((ROWS_PER_SC * 2 * MERGE_STRIDE,), jnp.float32),
        pltpu.VMEM_SHARED((ROWS_PER_SC * 2 * REC_STRIDE,), jnp.float32),
        pltpu.VMEM_SHARED((ROWS_PER_SC * 2 * REC_STRIDE,), jnp.int32),
        pltpu.VMEM_SHARED((ROWS_PER_SC * 2 * REC_STRIDE,), jnp.int32),
        pltpu.VMEM_SHARED((ROWS_PER_SC * 2 * REC_STRIDE,), jnp.float32),
    ],
)(_sc_body)


def kernel(input_ids, logits, input_vector, random_vectors):
    del input_ids
    key0 = jax.random.key(SEED)
    powers = 2 ** jnp.arange(BBITS, dtype=jnp.int32)

    def row_hash(vec):
        proj = random_vectors @ vec
        bits = (proj > 0).astype(jnp.int32)
        return jnp.sum(bits * powers)

    hashes = jax.vmap(row_hash)(input_vector)
    keys = jax.vmap(lambda h: jax.random.fold_in(key0, h))(hashes)
    xi = jax.vmap(lambda k: jax.random.uniform(
        k, (VOCAB,), dtype=jnp.float32, minval=1e-9, maxval=1.0))(keys)
    nlx = -jnp.log(xi)
    flat = _sc_call(logits.reshape(-1), nlx.reshape(-1))
    return flat.reshape(B, VOCAB)


# speedup vs baseline: 2.7736x; 1.0249x over previous
import functools

import jax
import jax.numpy as jnp
from jax import lax
from jax.experimental import pallas as pl
from jax.experimental.pallas import tpu as pltpu
from jax.experimental.pallas import tpu_sc as plsc

B = 16
VOCAB = 1000000
EMBED = 768
BBITS = 16
SEED = 42
TOP_P = 0.9

HALF = VOCAB // 2
CHUNK = 10000
NCHUNK = HALF // CHUNK
NVEC = CHUNK // 16
NPAIR = NCHUNK // 2
NB1 = 4096
NB2 = 4096
NB3 = 256
HWORDS = NB1 * 16
TIE_CAP = 64
TIE_WORDS = TIE_CAP + 16
FILL = -100000.0
MARK = 100000.0
ROWS_PER_SC = 8
MERGE_CHUNK = 10000
MERGE_STRIDE = 10112
REC_STRIDE = 128

_LANE = None


def _keys_of(x):
    u = plsc.bitcast(x, jnp.uint32)
    s = u >> jnp.uint32(31)
    flip = (jnp.uint32(0) - s) | jnp.uint32(0x80000000)
    return u ^ flip


def _sc_body(logits_hbm, nlx_hbm, out_hbm,
             xbuf, nbuf, xbuf2, nbuf2, hist, tieidx, tienlx,
             recfv, reciv, patch, sem0, sem1, sem2, sem3,
             hist_sh, recf_sh, reci_sh, tidx_sh, tnlx_sh):
    c = lax.axis_index("c")
    s = lax.axis_index("s")
    row_l = s // 2
    half = s % 2
    row = c * ROWS_PER_SC + row_l
    base = half * HALF
    rbase = row * VOCAB + base

    lane = jnp.arange(16, dtype=jnp.int32)
    zero16 = jnp.zeros((16,), jnp.float32)
    inf16 = jnp.full((16,), jnp.inf, dtype=jnp.float32)

    def exf(vec, n):
        return jnp.sum(jnp.where(lane == n, vec, 0.0))

    def exi(vec, n):
        return jnp.sum(jnp.where(lane == n, vec, 0))

    def zero_hist():
        def bd(i, carry):
            hist[pl.ds(i * 16, 16)] = zero16
            return carry
        lax.fori_loop(0, HWORDS // 16, bd, 0)

    def hist_pass(shift, nbits, prefix_shift, prefix):
        bmask = jnp.uint32((1 << nbits) - 1)
        shift_u = jnp.uint32(shift)

        def compute(buf):
            def vec_body(vi, c2):
                x = buf[pl.ds(vi * 16, 16)]
                w = jnp.exp(x)
                k = _keys_of(x)
                b = ((k >> shift_u) & bmask).astype(jnp.int32)
                addr = b * 16 + lane
                if prefix_shift is None:
                    plsc.addupdate_scatter(hist, [addr], w)
                else:
                    pred = (k >> jnp.uint32(prefix_shift)) == prefix
                    plsc.addupdate_scatter(hist, [addr], w, mask=pred)
                return c2
            lax.fori_loop(0, NVEC, vec_body, 0)

        def src_at(ci):
            return logits_hbm.at[pl.ds(rbase + ci * CHUNK, CHUNK)]

        pltpu.async_copy(src_at(0), xbuf, sem0)

        def pair(p, carry):
            pltpu.async_copy(src_at(2 * p + 1), xbuf2, sem1)
            pltpu.make_async_copy(src_at(2 * p), xbuf, sem0).wait()
            compute(xbuf)

            @pl.when(p < NPAIR - 1)
            def _pref():
                pltpu.async_copy(src_at(2 * p + 2), xbuf, sem0)
            pltpu.make_async_copy(src_at(2 * p + 1), xbuf2, sem1).wait()
            compute(xbuf2)
            return carry
        lax.fori_loop(0, NPAIR, pair, 0)

    def merge_hist():
        slot = (row_l * 2 + half) * MERGE_STRIDE
        oslot = (row_l * 2 + (1 - half)) * MERGE_STRIDE
        total = zero16
        for off, ln in ((0, 10000), (10000, 10000), (20000, 10000),
                        (30000, 10000), (40000, 10000), (50000, 10000),
                        (60000, 5536)):
            pltpu.sync_copy(hist.at[pl.ds(off, ln)],
                            hist_sh.at[pl.ds(slot, ln)])
            plsc.subcore_barrier()
            pltpu.sync_copy(hist_sh.at[pl.ds(oslot, ln)],
                            nbuf.at[pl.ds(0, ln)])

            def vb(vi, tot):
                cur = hist[pl.ds(off + vi * 16, 16)]
                oth = nbuf[pl.ds(vi * 16, 16)]
                m = cur + oth
                hist[pl.ds(off + vi * 16, 16)] = m
                return tot + m
            total = lax.fori_loop(0, ln // 16, vb, total)
            plsc.subcore_barrier()
        return jnp.sum(total)

    def scan_level(nbuckets, thresh, base_mass):
        def cond(st):
            j, acc, found, bst, m0, ln_b, ln_m0 = st
            return jnp.logical_and(j >= 0, jnp.logical_not(found))

        def body(st):
            j, acc, found, bst, m0, ln_b, ln_m0 = st
            bm = jnp.sum(hist[pl.ds(j * 16, 16)])
            acc2 = acc + bm
            crossed = acc2 >= thresh
            nonempty = bm > 0.0
            return (j - 1, acc2, crossed,
                    jnp.where(crossed, j, bst),
                    jnp.where(crossed, acc, m0),
                    jnp.where(nonempty, j, ln_b),
                    jnp.where(nonempty, acc, ln_m0))

        init = (jnp.int32(nbuckets - 1), base_mass, False,
                jnp.int32(0), base_mass, jnp.int32(0), base_mass)
        j, acc, found, bst, m0, ln_b, ln_m0 = lax.while_loop(
            cond, body, init)
        bst = jnp.where(found, bst, ln_b)
        m0 = jnp.where(found, m0, ln_m0)
        return bst, m0

    zero_hist()
    hist_pass(20, 12, None, None)
    e_total = merge_hist()
    thresh = jnp.float32(TOP_P) * e_total
    b1, m0_1 = scan_level(NB1, thresh, jnp.float32(0.0))
    b1u = b1.astype(jnp.uint32)

    zero_hist()
    hist_pass(8, 12, 20, b1u)
    merge_hist()
    b2, m0_2 = scan_level(NB2, thresh, m0_1)
    b2u = b2.astype(jnp.uint32)

    zero_hist()
    pref24 = (b1u << jnp.uint32(12)) | b2u
    hist_pass(0, 8, 8, pref24)
    merge_hist()
    b3, m0_3 = scan_level(NB3, thresh, m0_2)
    b3u = b3.astype(jnp.uint32)

    kstar = (b1u << jnp.uint32(20)) | (b2u << jnp.uint32(8)) | b3u
    m0 = m0_3

    kv = jnp.broadcast_to(kstar, (16,))
    sbit = kv >> jnp.uint32(31)
    uv = jnp.where(sbit == jnp.uint32(1),
                   kv ^ jnp.uint32(0x80000000), ~kv)
    wv_vec = jnp.exp(plsc.bitcast(uv, jnp.float32))
    w_v = jnp.max(wv_vec)

    qr_vec = jnp.broadcast_to(thresh - m0, (16,)) / wv_vec
    qr = jnp.minimum(jnp.max(qr_vec), jnp.float32(1.5e9))
    r0 = qr.astype(jnp.int32)
    bump = (r0.astype(jnp.float32) * w_v + m0 < thresh).astype(jnp.int32)
    r = jnp.maximum(r0 + bump, 1)

    def sel_compute(xb, nb, ci, st):
        def vec_body(vi, st2):
            best, bidx, cnt, manl, maix = st2
            x = xb[pl.ds(vi * 16, 16)]
            nl = nb[pl.ds(vi * 16, 16)]
            w = jnp.exp(x)
            k = _keys_of(x)
            idxv = base + ci * CHUNK + vi * 16 + lane
            certain = k > kstar
            sc = jnp.where(certain, nl / w, jnp.inf)
            better = sc < best
            best = jnp.where(better, sc, best)
            bidx = jnp.where(better, idxv, bidx)
            tiem = k == kstar
            offs = jnp.minimum(jnp.max(cnt), TIE_CAP)
            plsc.store_compressed(tieidx.at[pl.ds(offs, 16)], idxv,
                                  mask=tiem)
            tnl = jnp.where(tiem, nl, jnp.inf)
            plsc.store_compressed(tienlx.at[pl.ds(offs, 16)], tnl,
                                  mask=tiem)
            cnt = cnt + plsc.all_reduce_population_count(tiem)
            tbet = tnl < manl
            manl = jnp.where(tbet, tnl, manl)
            maix = jnp.where(tbet, idxv, maix)
            return best, bidx, cnt, manl, maix

        return lax.fori_loop(0, NVEC, vec_body, st)

    def xsrc(ci):
        return logits_hbm.at[pl.ds(rbase + ci * CHUNK, CHUNK)]

    def nsrc(ci):
        return nlx_hbm.at[pl.ds(rbase + ci * CHUNK, CHUNK)]

    pltpu.async_copy(xsrc(0), xbuf, sem0)
    pltpu.async_copy(nsrc(0), nbuf, sem2)

    def sel_pair(p, st):
        pltpu.async_copy(xsrc(2 * p + 1), xbuf2, sem1)
        pltpu.async_copy(nsrc(2 * p + 1), nbuf2, sem3)
        pltpu.make_async_copy(xsrc(2 * p), xbuf, sem0).wait()
        pltpu.make_async_copy(nsrc(2 * p), nbuf, sem2).wait()
        st = sel_compute(xbuf, nbuf, 2 * p, st)

        @pl.when(p < NPAIR - 1)
        def _pref():
            pltpu.async_copy(xsrc(2 * p + 2), xbuf, sem0)
            pltpu.async_copy(nsrc(2 * p + 2), nbuf, sem2)
        pltpu.make_async_copy(xsrc(2 * p + 1), xbuf2, sem1).wait()
        pltpu.make_async_copy(nsrc(2 * p + 1), nbuf2, sem3).wait()
        st = sel_compute(xbuf2, nbuf2, 2 * p + 1, st)
        return st

    izero = jnp.zeros((16,), jnp.int32)
    best, bidx, cnt, manl, maix = lax.fori_loop(
        0, NPAIR, sel_pair, (inf16, izero, izero, inf16, izero))

    bigi = jnp.int32(2 ** 30)
    bestm = jnp.min(best)
    besti = jnp.min(jnp.where(best == bestm, bidx, bigi))
    ncnt = jnp.max(cnt)
    manl_s = jnp.min(manl)
    maix_s = jnp.min(jnp.where(manl == manl_s, maix, bigi))

    recf = jnp.where(lane == 0, bestm, jnp.where(lane == 1, manl_s, 0.0))
    reci = jnp.where(lane == 0, besti,
                     jnp.where(lane == 1, ncnt,
                               jnp.where(lane == 2, maix_s, 0)))
    recfv[...] = recf
    reciv[...] = reci
    myslot = row_l * 2 + half
    pltpu.sync_copy(recfv, recf_sh.at[pl.ds(myslot * REC_STRIDE, 16)])
    pltpu.sync_copy(reciv, reci_sh.at[pl.ds(myslot * REC_STRIDE, 16)])
    pltpu.sync_copy(tieidx, tidx_sh.at[pl.ds(myslot * REC_STRIDE, TIE_WORDS)])
    pltpu.sync_copy(tienlx, tnlx_sh.at[pl.ds(myslot * REC_STRIDE, TIE_WORDS)])
    plsc.subcore_barrier()

    s0 = row_l * 2
    pltpu.sync_copy(recf_sh.at[pl.ds(s0 * REC_STRIDE, 16)], recfv)
    pltpu.sync_copy(reci_sh.at[pl.ds(s0 * REC_STRIDE, 16)], reciv)
    rf0 = recfv[...]
    ri0 = reciv[...]
    bm0 = exf(rf0, 0)
    manl0 = exf(rf0, 1)
    bi0 = exi(ri0, 0)
    n0 = exi(ri0, 1)
    mai0 = exi(ri0, 2)
    s1 = row_l * 2 + 1
    pltpu.sync_copy(recf_sh.at[pl.ds(s1 * REC_STRIDE, 16)], recfv)
    pltpu.sync_copy(reci_sh.at[pl.ds(s1 * REC_STRIDE, 16)], reciv)
    rf1 = recfv[...]
    ri1 = reciv[...]
    bm1 = exf(rf1, 0)
    manl1 = exf(rf1, 1)
    bi1 = exi(ri1, 0)
    n1 = exi(ri1, 1)
    mai1 = exi(ri1, 2)

    k0 = jnp.minimum(r, n0)
    k1 = jnp.minimum(jnp.maximum(r - n0, 0), n1)
    use_all = jnp.logical_or(r >= n0 + n1,
                             jnp.logical_or(k0 > TIE_CAP, k1 > TIE_CAP))

    def walk(hh, kk, st):
        hs = (row_l * 2 + hh) * REC_STRIDE
        pltpu.sync_copy(tidx_sh.at[pl.ds(hs, TIE_WORDS)], tieidx)
        pltpu.sync_copy(tnlx_sh.at[pl.ds(hs, TIE_WORDS)], tienlx)

        def wb(j, st2):
            mn, mi = st2
            lm = (j * 16 + lane) < kk
            vals = jnp.where(lm, tienlx[pl.ds(j * 16, 16)], jnp.inf)
            ids = tieidx[pl.ds(j * 16, 16)]
            bet = vals < mn
            return jnp.where(bet, vals, mn), jnp.where(bet, ids, mi)
        return lax.fori_loop(0, TIE_CAP // 16, wb, st)

    tmn, tmi = walk(0, k0, (inf16, izero))
    tmn, tmi = walk(1, k1, (tmn, tmi))
    t_nl = jnp.min(tmn)
    t_ix = jnp.min(jnp.where(tmn == t_nl, tmi, bigi))

    all_nl = jnp.where(manl0 <= manl1, manl0, manl1)
    all_ix = jnp.where(manl0 <= manl1, mai0, mai1)
    tie_nl = jnp.where(use_all, all_nl, t_nl)
    tie_ix = jnp.where(use_all, all_ix, t_ix)
    tie_sc = jnp.max(jnp.broadcast_to(tie_nl, (16,)) / wv_vec)

    ws = bm0
    wi = bi0
    upd = bm1 < ws
    ws = jnp.where(upd, bm1, ws)
    wi = jnp.where(upd, bi1, wi)
    updt = tie_sc < ws
    wi = jnp.where(updt, tie_ix, wi)

    fill16 = jnp.full((16,), FILL, dtype=jnp.float32)

    def fb(i, carry):
        hist[pl.ds(i * 16, 16)] = fill16
        return carry
    lax.fori_loop(0, HWORDS // 16, fb, 0)
    rem = HALF - 7 * HWORDS
    for i in range(7):
        pltpu.async_copy(hist,
                         out_hbm.at[pl.ds(rbase + i * HWORDS, HWORDS)],
                         sem0)
    pltpu.async_copy(hist.at[pl.ds(0, rem)],
                     out_hbm.at[pl.ds(rbase + 7 * HWORDS, rem)], sem0)
    for i in range(7):
        pltpu.make_async_copy(
            hist, out_hbm.at[pl.ds(rbase + i * HWORDS, HWORDS)],
            sem0).wait()
    pltpu.make_async_copy(
        hist.at[pl.ds(0, rem)],
        out_hbm.at[pl.ds(rbase + 7 * HWORDS, rem)], sem0).wait()

    own = jnp.logical_and(wi >= base, wi < base + HALF)

    @pl.when(own)
    def _patch():
        pb = wi & jnp.int32(-16)
        off = wi - pb
        patch[...] = jnp.where(lane == off, MARK, FILL)
        dst = pl.multiple_of(row * VOCAB + pb, 16)
        pltpu.sync_copy(patch, out_hbm.at[pl.ds(dst, 16)])


_mesh = plsc.VectorSubcoreMesh(core_axis_name="c", subcore_axis_name="s",
                               num_cores=2, num_subcores=16)

_sc_call = functools.partial(
    pl.kernel,
    out_type=jax.ShapeDtypeStruct((B * VOCAB,), jnp.float32),
    mesh=_mesh,
    compiler_params=pltpu.CompilerParams(needs_layout_passes=False),
    scratch_types=[
        pltpu.VMEM((CHUNK,), jnp.float32),
        pltpu.VMEM((CHUNK,), jnp.float32),
        pltpu.VMEM((CHUNK,), jnp.float32),
        pltpu.VMEM((CHUNK,), jnp.float32),
        pltpu.VMEM((HWORDS,), jnp.float32),
        pltpu.VMEM((TIE_WORDS,), jnp.int32),
        pltpu.VMEM((TIE_WORDS,), jnp.float32),
        pltpu.VMEM((16,), jnp.float32),
        pltpu.VMEM((16,), jnp.int32),
        pltpu.VMEM((16,), jnp.float32),
        pltpu.SemaphoreType.DMA,
        pltpu.SemaphoreType.DMA,
        pltpu.SemaphoreType.DMA,
        pltpu.SemaphoreType.DMA,
        pltpu.VMEM_SHARED((ROWS_PER_SC * 2 * MERGE_STRIDE,), jnp.float32),
        pltpu.VMEM_SHARED((ROWS_PER_SC * 2 * REC_STRIDE,), jnp.float32),
        pltpu.VMEM_SHARED((ROWS_PER_SC * 2 * REC_STRIDE,), jnp.int32),
        pltpu.VMEM_SHARED((ROWS_PER_SC * 2 * REC_STRIDE,), jnp.int32),
        pltpu.VMEM_SHARED((ROWS_PER_SC * 2 * REC_STRIDE,), jnp.float32),
    ],
)(_sc_body)


def kernel(input_ids, logits, input_vector, random_vectors):
    del input_ids
    key0 = jax.random.key(SEED)
    powers = 2 ** jnp.arange(BBITS, dtype=jnp.int32)

    def row_hash(vec):
        proj = random_vectors @ vec
        bits = (proj > 0).astype(jnp.int32)
        return jnp.sum(bits * powers)

    hashes = jax.vmap(row_hash)(input_vector)
    keys = jax.vmap(lambda h: jax.random.fold_in(key0, h))(hashes)
    xi = jax.vmap(lambda k: jax.random.uniform(
        k, (VOCAB,), dtype=jnp.float32, minval=1e-9, maxval=1.0))(keys)
    nlx = -jnp.log(xi)
    flat = _sc_call(logits.reshape(-1), nlx.reshape(-1))
    return flat.reshape(B, VOCAB)


# speedup vs baseline: 2.8118x; 1.0138x over previous
import functools

import jax
import jax.numpy as jnp
from jax import lax
from jax.experimental import pallas as pl
from jax.experimental.pallas import tpu as pltpu
from jax.experimental.pallas import tpu_sc as plsc

B = 16
VOCAB = 1000000
EMBED = 768
BBITS = 16
SEED = 42
TOP_P = 0.9

HALF = VOCAB // 2
CHUNK = 10000
NCHUNK = HALF // CHUNK
NVEC = CHUNK // 16
NPAIR = NCHUNK // 2
UNROLL = 5
NB1 = 4096
NB2 = 4096
NB3 = 256
HWORDS = NB1 * 16
TIE_CAP = 64
TIE_WORDS = TIE_CAP + 16
FILL = -100000.0
MARK = 100000.0
ROWS_PER_SC = 8
MERGE_CHUNK = 10000
MERGE_STRIDE = 10112
REC_STRIDE = 128

_LANE = None


def _keys_of(x):
    u = plsc.bitcast(x, jnp.uint32)
    s = u >> jnp.uint32(31)
    flip = (jnp.uint32(0) - s) | jnp.uint32(0x80000000)
    return u ^ flip


def _sc_body(logits_hbm, nlx_hbm, out_hbm,
             xbuf, nbuf, xbuf2, nbuf2, hist, tieidx, tienlx,
             recfv, reciv, patch, sem0, sem1, sem2, sem3,
             hist_sh, recf_sh, reci_sh, tidx_sh, tnlx_sh):
    c = lax.axis_index("c")
    s = lax.axis_index("s")
    row_l = s // 2
    half = s % 2
    row = c * ROWS_PER_SC + row_l
    base = half * HALF
    rbase = row * VOCAB + base

    lane = jnp.arange(16, dtype=jnp.int32)
    zero16 = jnp.zeros((16,), jnp.float32)
    inf16 = jnp.full((16,), jnp.inf, dtype=jnp.float32)

    def exf(vec, n):
        return jnp.sum(jnp.where(lane == n, vec, 0.0))

    def exi(vec, n):
        return jnp.sum(jnp.where(lane == n, vec, 0))

    def zero_hist():
        def bd(i, carry):
            for u in range(8):
                hist[pl.ds(i * 128 + u * 16, 16)] = zero16
            return carry
        lax.fori_loop(0, HWORDS // 128, bd, 0)

    def hist_pass(shift, nbits, prefix_shift, prefix):
        bmask = jnp.uint32((1 << nbits) - 1)
        shift_u = jnp.uint32(shift)

        def compute(buf):
            def vec_body(vi, c2):
                for u in range(UNROLL):
                    x = buf[pl.ds(vi * (16 * UNROLL) + u * 16, 16)]
                    w = jnp.exp(x)
                    k = _keys_of(x)
                    b = ((k >> shift_u) & bmask).astype(jnp.int32)
                    addr = b * 16 + lane
                    if prefix_shift is None:
                        plsc.addupdate_scatter(hist, [addr], w)
                    else:
                        pred = (k >> jnp.uint32(prefix_shift)) == prefix
                        plsc.addupdate_scatter(hist, [addr], w, mask=pred)
                return c2
            lax.fori_loop(0, NVEC // UNROLL, vec_body, 0)

        def src_at(ci):
            return logits_hbm.at[pl.ds(rbase + ci * CHUNK, CHUNK)]

        pltpu.async_copy(src_at(0), xbuf, sem0)

        def pair(p, carry):
            pltpu.async_copy(src_at(2 * p + 1), xbuf2, sem1)
            pltpu.make_async_copy(src_at(2 * p), xbuf, sem0).wait()
            compute(xbuf)

            @pl.when(p < NPAIR - 1)
            def _pref():
                pltpu.async_copy(src_at(2 * p + 2), xbuf, sem0)
            pltpu.make_async_copy(src_at(2 * p + 1), xbuf2, sem1).wait()
            compute(xbuf2)
            return carry
        lax.fori_loop(0, NPAIR, pair, 0)

    def merge_hist():
        slot = (row_l * 2 + half) * MERGE_STRIDE
        oslot = (row_l * 2 + (1 - half)) * MERGE_STRIDE
        total = zero16
        for off, ln in ((0, 10000), (10000, 10000), (20000, 10000),
                        (30000, 10000), (40000, 10000), (50000, 10000),
                        (60000, 5536)):
            pltpu.sync_copy(hist.at[pl.ds(off, ln)],
                            hist_sh.at[pl.ds(slot, ln)])
            plsc.subcore_barrier()
            pltpu.sync_copy(hist_sh.at[pl.ds(oslot, ln)],
                            nbuf.at[pl.ds(0, ln)])

            if ln % (16 * UNROLL) == 0:
                def vb(vi, tot):
                    for u in range(UNROLL):
                        o2 = vi * (16 * UNROLL) + u * 16
                        m = hist[pl.ds(off + o2, 16)] + nbuf[pl.ds(o2, 16)]
                        hist[pl.ds(off + o2, 16)] = m
                        tot = tot + m
                    return tot
                total = lax.fori_loop(0, ln // (16 * UNROLL), vb, total)
            else:
                def vb(vi, tot):
                    m = hist[pl.ds(off + vi * 16, 16)] + nbuf[pl.ds(vi * 16, 16)]
                    hist[pl.ds(off + vi * 16, 16)] = m
                    return tot + m
                total = lax.fori_loop(0, ln // 16, vb, total)
            plsc.subcore_barrier()
        return jnp.sum(total)

    def scan_level(nbuckets, thresh, base_mass):
        def cond(st):
            j, acc, found, bst, m0, ln_b, ln_m0 = st
            return jnp.logical_and(j >= 0, jnp.logical_not(found))

        def body(st):
            j, acc, found, bst, m0, ln_b, ln_m0 = st
            bm = jnp.sum(hist[pl.ds(j * 16, 16)])
            acc2 = acc + bm
            crossed = acc2 >= thresh
            nonempty = bm > 0.0
            return (j - 1, acc2, crossed,
                    jnp.where(crossed, j, bst),
                    jnp.where(crossed, acc, m0),
                    jnp.where(nonempty, j, ln_b),
                    jnp.where(nonempty, acc, ln_m0))

        init = (jnp.int32(nbuckets - 1), base_mass, False,
                jnp.int32(0), base_mass, jnp.int32(0), base_mass)
        j, acc, found, bst, m0, ln_b, ln_m0 = lax.while_loop(
            cond, body, init)
        bst = jnp.where(found, bst, ln_b)
        m0 = jnp.where(found, m0, ln_m0)
        return bst, m0

    zero_hist()
    hist_pass(20, 12, None, None)
    e_total = merge_hist()
    thresh = jnp.float32(TOP_P) * e_total
    b1, m0_1 = scan_level(NB1, thresh, jnp.float32(0.0))
    b1u = b1.astype(jnp.uint32)

    zero_hist()
    hist_pass(8, 12, 20, b1u)
    merge_hist()
    b2, m0_2 = scan_level(NB2, thresh, m0_1)
    b2u = b2.astype(jnp.uint32)

    zero_hist()
    pref24 = (b1u << jnp.uint32(12)) | b2u
    hist_pass(0, 8, 8, pref24)
    merge_hist()
    b3, m0_3 = scan_level(NB3, thresh, m0_2)
    b3u = b3.astype(jnp.uint32)

    kstar = (b1u << jnp.uint32(20)) | (b2u << jnp.uint32(8)) | b3u
    m0 = m0_3

    kv = jnp.broadcast_to(kstar, (16,))
    sbit = kv >> jnp.uint32(31)
    uv = jnp.where(sbit == jnp.uint32(1),
                   kv ^ jnp.uint32(0x80000000), ~kv)
    wv_vec = jnp.exp(plsc.bitcast(uv, jnp.float32))
    w_v = jnp.max(wv_vec)

    qr_vec = jnp.broadcast_to(thresh - m0, (16,)) / wv_vec
    qr = jnp.minimum(jnp.max(qr_vec), jnp.float32(1.5e9))
    r0 = qr.astype(jnp.int32)
    bump = (r0.astype(jnp.float32) * w_v + m0 < thresh).astype(jnp.int32)
    r = jnp.maximum(r0 + bump, 1)

    def sel_compute(xb, nb, ci, st):
        def vec_body(vi, st2):
            best, bidx, cnt, manl, maix = st2
            for u in range(UNROLL):
                x = xb[pl.ds(vi * (16 * UNROLL) + u * 16, 16)]
                nl = nb[pl.ds(vi * (16 * UNROLL) + u * 16, 16)]
                w = jnp.exp(x)
                k = _keys_of(x)
                idxv = (base + ci * CHUNK + vi * (16 * UNROLL) + u * 16
                        + lane)
                certain = k > kstar
                sc = jnp.where(certain, nl / w, jnp.inf)
                better = sc < best
                best = jnp.where(better, sc, best)
                bidx = jnp.where(better, idxv, bidx)
                tiem = k == kstar
                offs = jnp.minimum(jnp.max(cnt), TIE_CAP)
                plsc.store_compressed(tieidx.at[pl.ds(offs, 16)], idxv,
                                      mask=tiem)
                tnl = jnp.where(tiem, nl, jnp.inf)
                plsc.store_compressed(tienlx.at[pl.ds(offs, 16)], tnl,
                                      mask=tiem)
                cnt = cnt + plsc.all_reduce_population_count(tiem)
                tbet = tnl < manl
                manl = jnp.where(tbet, tnl, manl)
                maix = jnp.where(tbet, idxv, maix)
            return best, bidx, cnt, manl, maix

        return lax.fori_loop(0, NVEC // UNROLL, vec_body, st)

    def xsrc(ci):
        return logits_hbm.at[pl.ds(rbase + ci * CHUNK, CHUNK)]

    def nsrc(ci):
        return nlx_hbm.at[pl.ds(rbase + ci * CHUNK, CHUNK)]

    pltpu.async_copy(xsrc(0), xbuf, sem0)
    pltpu.async_copy(nsrc(0), nbuf, sem2)

    def sel_pair(p, st):
        pltpu.async_copy(xsrc(2 * p + 1), xbuf2, sem1)
        pltpu.async_copy(nsrc(2 * p + 1), nbuf2, sem3)
        pltpu.make_async_copy(xsrc(2 * p), xbuf, sem0).wait()
        pltpu.make_async_copy(nsrc(2 * p), nbuf, sem2).wait()
        st = sel_compute(xbuf, nbuf, 2 * p, st)

        @pl.when(p < NPAIR - 1)
        def _pref():
            pltpu.async_copy(xsrc(2 * p + 2), xbuf, sem0)
            pltpu.async_copy(nsrc(2 * p + 2), nbuf, sem2)
        pltpu.make_async_copy(xsrc(2 * p + 1), xbuf2, sem1).wait()
        pltpu.make_async_copy(nsrc(2 * p + 1), nbuf2, sem3).wait()
        st = sel_compute(xbuf2, nbuf2, 2 * p + 1, st)
        return st

    izero = jnp.zeros((16,), jnp.int32)
    best, bidx, cnt, manl, maix = lax.fori_loop(
        0, NPAIR, sel_pair, (inf16, izero, izero, inf16, izero))

    bigi = jnp.int32(2 ** 30)
    bestm = jnp.min(best)
    besti = jnp.min(jnp.where(best == bestm, bidx, bigi))
    ncnt = jnp.max(cnt)
    manl_s = jnp.min(manl)
    maix_s = jnp.min(jnp.where(manl == manl_s, maix, bigi))

    recf = jnp.where(lane == 0, bestm, jnp.where(lane == 1, manl_s, 0.0))
    reci = jnp.where(lane == 0, besti,
                     jnp.where(lane == 1, ncnt,
                               jnp.where(lane == 2, maix_s, 0)))
    recfv[...] = recf
    reciv[...] = reci
    myslot = row_l * 2 + half
    pltpu.sync_copy(recfv, recf_sh.at[pl.ds(myslot * REC_STRIDE, 16)])
    pltpu.sync_copy(reciv, reci_sh.at[pl.ds(myslot * REC_STRIDE, 16)])
    pltpu.sync_copy(tieidx, tidx_sh.at[pl.ds(myslot * REC_STRIDE, TIE_WORDS)])
    pltpu.sync_copy(tienlx, tnlx_sh.at[pl.ds(myslot * REC_STRIDE, TIE_WORDS)])
    plsc.subcore_barrier()

    s0 = row_l * 2
    pltpu.sync_copy(recf_sh.at[pl.ds(s0 * REC_STRIDE, 16)], recfv)
    pltpu.sync_copy(reci_sh.at[pl.ds(s0 * REC_STRIDE, 16)], reciv)
    rf0 = recfv[...]
    ri0 = reciv[...]
    bm0 = exf(rf0, 0)
    manl0 = exf(rf0, 1)
    bi0 = exi(ri0, 0)
    n0 = exi(ri0, 1)
    mai0 = exi(ri0, 2)
    s1 = row_l * 2 + 1
    pltpu.sync_copy(recf_sh.at[pl.ds(s1 * REC_STRIDE, 16)], recfv)
    pltpu.sync_copy(reci_sh.at[pl.ds(s1 * REC_STRIDE, 16)], reciv)
    rf1 = recfv[...]
    ri1 = reciv[...]
    bm1 = exf(rf1, 0)
    manl1 = exf(rf1, 1)
    bi1 = exi(ri1, 0)
    n1 = exi(ri1, 1)
    mai1 = exi(ri1, 2)

    k0 = jnp.minimum(r, n0)
    k1 = jnp.minimum(jnp.maximum(r - n0, 0), n1)
    use_all = jnp.logical_or(r >= n0 + n1,
                             jnp.logical_or(k0 > TIE_CAP, k1 > TIE_CAP))

    def walk(hh, kk, st):
        hs = (row_l * 2 + hh) * REC_STRIDE
        pltpu.sync_copy(tidx_sh.at[pl.ds(hs, TIE_WORDS)], tieidx)
        pltpu.sync_copy(tnlx_sh.at[pl.ds(hs, TIE_WORDS)], tienlx)

        def wb(j, st2):
            mn, mi = st2
            lm = (j * 16 + lane) < kk
            vals = jnp.where(lm, tienlx[pl.ds(j * 16, 16)], jnp.inf)
            ids = tieidx[pl.ds(j * 16, 16)]
            bet = vals < mn
            return jnp.where(bet, vals, mn), jnp.where(bet, ids, mi)
        return lax.fori_loop(0, TIE_CAP // 16, wb, st)

    tmn, tmi = walk(0, k0, (inf16, izero))
    tmn, tmi = walk(1, k1, (tmn, tmi))
    t_nl = jnp.min(tmn)
    t_ix = jnp.min(jnp.where(tmn == t_nl, tmi, bigi))

    all_nl = jnp.where(manl0 <= manl1, manl0, manl1)
    all_ix = jnp.where(manl0 <= manl1, mai0, mai1)
    tie_nl = jnp.where(use_all, all_nl, t_nl)
    tie_ix = jnp.where(use_all, all_ix, t_ix)
    tie_sc = jnp.max(jnp.broadcast_to(tie_nl, (16,)) / wv_vec)

    ws = bm0
    wi = bi0
    upd = bm1 < ws
    ws = jnp.where(upd, bm1, ws)
    wi = jnp.where(upd, bi1, wi)
    updt = tie_sc < ws
    wi = jnp.where(updt, tie_ix, wi)

    fill16 = jnp.full((16,), FILL, dtype=jnp.float32)

    def fb(i, carry):
        for u in range(8):
            hist[pl.ds(i * 128 + u * 16, 16)] = fill16
        return carry
    lax.fori_loop(0, HWORDS // 128, fb, 0)
    rem = HALF - 7 * HWORDS
    for i in range(7):
        pltpu.async_copy(hist,
                         out_hbm.at[pl.ds(rbase + i * HWORDS, HWORDS)],
                         sem0)
    pltpu.async_copy(hist.at[pl.ds(0, rem)],
                     out_hbm.at[pl.ds(rbase + 7 * HWORDS, rem)], sem0)
    for i in range(7):
        pltpu.make_async_copy(
            hist, out_hbm.at[pl.ds(rbase + i * HWORDS, HWORDS)],
            sem0).wait()
    pltpu.make_async_copy(
        hist.at[pl.ds(0, rem)],
        out_hbm.at[pl.ds(rbase + 7 * HWORDS, rem)], sem0).wait()

    own = jnp.logical_and(wi >= base, wi < base + HALF)

    @pl.when(own)
    def _patch():
        pb = wi & jnp.int32(-16)
        off = wi - pb
        patch[...] = jnp.where(lane == off, MARK, FILL)
        dst = pl.multiple_of(row * VOCAB + pb, 16)
        pltpu.sync_copy(patch, out_hbm.at[pl.ds(dst, 16)])


_mesh = plsc.VectorSubcoreMesh(core_axis_name="c", subcore_axis_name="s",
                               num_cores=2, num_subcores=16)

_sc_call = functools.partial(
    pl.kernel,
    out_type=jax.ShapeDtypeStruct((B * VOCAB,), jnp.float32),
    mesh=_mesh,
    compiler_params=pltpu.CompilerParams(needs_layout_passes=False),
    scratch_types=[
        pltpu.VMEM((CHUNK,), jnp.float32),
        pltpu.VMEM((CHUNK,), jnp.float32),
        pltpu.VMEM((CHUNK,), jnp.float32),
        pltpu.VMEM((CHUNK,), jnp.float32),
        pltpu.VMEM((HWORDS,), jnp.float32),
        pltpu.VMEM((TIE_WORDS,), jnp.int32),
        pltpu.VMEM((TIE_WORDS,), jnp.float32),
        pltpu.VMEM((16,), jnp.float32),
        pltpu.VMEM((16,), jnp.int32),
        pltpu.VMEM((16,), jnp.float32),
        pltpu.SemaphoreType.DMA,
        pltpu.SemaphoreType.DMA,
        pltpu.SemaphoreType.DMA,
        pltpu.SemaphoreType.DMA,
        pltpu.VMEM_SHARED((ROWS_PER_SC * 2 * MERGE_STRIDE,), jnp.float32),
        pltpu.VMEM_SHARED((ROWS_PER_SC * 2 * REC_STRIDE,), jnp.float32),
        pltpu.VMEM_SHARED((ROWS_PER_SC * 2 * REC_STRIDE,), jnp.int32),
        pltpu.VMEM_SHARED((ROWS_PER_SC * 2 * REC_STRIDE,), jnp.int32),
        pltpu.VMEM_SHARED((ROWS_PER_SC * 2 * REC_STRIDE,), jnp.float32),
    ],
)(_sc_body)


def kernel(input_ids, logits, input_vector, random_vectors):
    del input_ids
    key0 = jax.random.key(SEED)
    powers = 2 ** jnp.arange(BBITS, dtype=jnp.int32)

    def row_hash(vec):
        proj = random_vectors @ vec
        bits = (proj > 0).astype(jnp.int32)
        return jnp.sum(bits * powers)

    hashes = jax.vmap(row_hash)(input_vector)
    keys = jax.vmap(lambda h: jax.random.fold_in(key0, h))(hashes)
    xi = jax.vmap(lambda k: jax.random.uniform(
        k, (VOCAB,), dtype=jnp.float32, minval=1e-9, maxval=1.0))(keys)
    nlx = -jnp.log(xi)
    flat = _sc_call(logits.reshape(-1), nlx.reshape(-1))
    return flat.reshape(B, VOCAB)


# speedup vs baseline: 2.8889x; 1.0274x over previous
import functools

import jax
import jax.numpy as jnp
from jax import lax
from jax.experimental import pallas as pl
from jax.experimental.pallas import tpu as pltpu
from jax.experimental.pallas import tpu_sc as plsc

B = 16
VOCAB = 1000000
EMBED = 768
BBITS = 16
SEED = 42
TOP_P = 0.9

HALF = VOCAB // 2
CHUNK = 10000
NCHUNK = HALF // CHUNK
NVEC = CHUNK // 16
NPAIR = NCHUNK // 2
UNROLL = 5
NB1 = 4096
NB2 = 4096
NB3 = 256
HWORDS = NB1 * 16
TIE_CAP = 64
TIE_WORDS = TIE_CAP + 16
FILL = -100000.0
MARK = 100000.0
ROWS_PER_SC = 8
MERGE_CHUNK = 10000
MERGE_STRIDE = 10112
REC_STRIDE = 128

_LANE = None


def _keys_of(x):
    u = plsc.bitcast(x, jnp.uint32)
    s = u >> jnp.uint32(31)
    flip = (jnp.uint32(0) - s) | jnp.uint32(0x80000000)
    return u ^ flip


def _sc_body(logits_hbm, nlx_hbm, out_hbm,
             xbuf, nbuf, xbuf2, nbuf2, hist, tieidx, tienlx,
             recfv, reciv, patch, sem0, sem1, sem2, sem3,
             hist_sh, recf_sh, reci_sh, tidx_sh, tnlx_sh):
    c = lax.axis_index("c")
    s = lax.axis_index("s")
    row_l = s // 2
    half = s % 2
    row = c * ROWS_PER_SC + row_l
    base = half * HALF
    rbase = row * VOCAB + base

    lane = jnp.arange(16, dtype=jnp.int32)
    zero16 = jnp.zeros((16,), jnp.float32)
    inf16 = jnp.full((16,), jnp.inf, dtype=jnp.float32)

    def exf(vec, n):
        return jnp.sum(jnp.where(lane == n, vec, 0.0))

    def exi(vec, n):
        return jnp.sum(jnp.where(lane == n, vec, 0))

    def zero_hist():
        def bd(i, carry):
            for u in range(8):
                hist[pl.ds(i * 128 + u * 16, 16)] = zero16
            return carry
        lax.fori_loop(0, HWORDS // 128, bd, 0)

    def hist_pass(shift, nbits, prefix_shift, prefix):
        bmask = jnp.uint32((1 << nbits) - 1)
        shift_u = jnp.uint32(shift)

        def compute(buf):
            def vec_body(vi, c2):
                for u in range(UNROLL):
                    x = buf[pl.ds(vi * (16 * UNROLL) + u * 16, 16)]
                    w = jnp.exp(x)
                    k = _keys_of(x)
                    b = ((k >> shift_u) & bmask).astype(jnp.int32)
                    addr = b * 16 + lane
                    if prefix_shift is None:
                        plsc.addupdate_scatter(hist, [addr], w)
                    else:
                        pred = (k >> jnp.uint32(prefix_shift)) == prefix
                        plsc.addupdate_scatter(hist, [addr], w, mask=pred)
                return c2
            lax.fori_loop(0, NVEC // UNROLL, vec_body, 0)

        def src_at(ci):
            return logits_hbm.at[pl.ds(rbase + ci * CHUNK, CHUNK)]

        pltpu.async_copy(src_at(0), xbuf, sem0)

        def pair(p, carry):
            pltpu.async_copy(src_at(2 * p + 1), xbuf2, sem1)
            pltpu.make_async_copy(src_at(2 * p), xbuf, sem0).wait()
            compute(xbuf)

            @pl.when(p < NPAIR - 1)
            def _pref():
                pltpu.async_copy(src_at(2 * p + 2), xbuf, sem0)
            pltpu.make_async_copy(src_at(2 * p + 1), xbuf2, sem1).wait()
            compute(xbuf2)
            return carry
        lax.fori_loop(0, NPAIR, pair, 0)

    def merge_hist():
        slot = (row_l * 2 + half) * MERGE_STRIDE
        oslot = (row_l * 2 + (1 - half)) * MERGE_STRIDE
        total = zero16
        for off, ln in ((0, 10000), (10000, 10000), (20000, 10000),
                        (30000, 10000), (40000, 10000), (50000, 10000),
                        (60000, 5536)):
            pltpu.sync_copy(hist.at[pl.ds(off, ln)],
                            hist_sh.at[pl.ds(slot, ln)])
            plsc.subcore_barrier()
            pltpu.sync_copy(hist_sh.at[pl.ds(oslot, ln)],
                            nbuf.at[pl.ds(0, ln)])

            if ln % (16 * UNROLL) == 0:
                def vb(vi, tot):
                    for u in range(UNROLL):
                        o2 = vi * (16 * UNROLL) + u * 16
                        m = hist[pl.ds(off + o2, 16)] + nbuf[pl.ds(o2, 16)]
                        hist[pl.ds(off + o2, 16)] = m
                        tot = tot + m
                    return tot
                total = lax.fori_loop(0, ln // (16 * UNROLL), vb, total)
            else:
                def vb(vi, tot):
                    m = hist[pl.ds(off + vi * 16, 16)] + nbuf[pl.ds(vi * 16, 16)]
                    hist[pl.ds(off + vi * 16, 16)] = m
                    return tot + m
                total = lax.fori_loop(0, ln // 16, vb, total)
            plsc.subcore_barrier()
        return jnp.sum(total)

    def scan_level(nbuckets, thresh, base_mass):
        def cond(st):
            j, acc, found, bst, m0, ln_b, ln_m0 = st
            return jnp.logical_and(j >= 0, jnp.logical_not(found))

        def body(st):
            j, acc, found, bst, m0, ln_b, ln_m0 = st
            bm = jnp.sum(hist[pl.ds(j * 16, 16)])
            acc2 = acc + bm
            crossed = acc2 >= thresh
            nonempty = bm > 0.0
            return (j - 1, acc2, crossed,
                    jnp.where(crossed, j, bst),
                    jnp.where(crossed, acc, m0),
                    jnp.where(nonempty, j, ln_b),
                    jnp.where(nonempty, acc, ln_m0))

        init = (jnp.int32(nbuckets - 1), base_mass, False,
                jnp.int32(0), base_mass, jnp.int32(0), base_mass)
        j, acc, found, bst, m0, ln_b, ln_m0 = lax.while_loop(
            cond, body, init)
        bst = jnp.where(found, bst, ln_b)
        m0 = jnp.where(found, m0, ln_m0)
        return bst, m0

    zero_hist()
    hist_pass(20, 12, None, None)
    e_total = merge_hist()
    thresh = jnp.float32(TOP_P) * e_total
    b1, m0_1 = scan_level(NB1, thresh, jnp.float32(0.0))
    b1u = b1.astype(jnp.uint32)

    zero_hist()
    hist_pass(8, 12, 20, b1u)
    merge_hist()
    b2, m0_2 = scan_level(NB2, thresh, m0_1)
    b2u = b2.astype(jnp.uint32)

    zero_hist()
    pref24 = (b1u << jnp.uint32(12)) | b2u
    hist_pass(0, 8, 8, pref24)
    merge_hist()
    b3, m0_3 = scan_level(NB3, thresh, m0_2)
    b3u = b3.astype(jnp.uint32)

    kstar = (b1u << jnp.uint32(20)) | (b2u << jnp.uint32(8)) | b3u
    m0 = m0_3

    kv = jnp.broadcast_to(kstar, (16,))
    sbit = kv >> jnp.uint32(31)
    uv = jnp.where(sbit == jnp.uint32(1),
                   kv ^ jnp.uint32(0x80000000), ~kv)
    wv_vec = jnp.exp(plsc.bitcast(uv, jnp.float32))
    w_v = jnp.max(wv_vec)

    qr_vec = jnp.broadcast_to(thresh - m0, (16,)) / wv_vec
    qr = jnp.minimum(jnp.max(qr_vec), jnp.float32(1.5e9))
    r0 = qr.astype(jnp.int32)
    bump = (r0.astype(jnp.float32) * w_v + m0 < thresh).astype(jnp.int32)
    r = jnp.maximum(r0 + bump, 1)

    def slow_ties(xb, nb, ci, cnt, manl, maix):
        def sb(vi, st3):
            cnt, manl, maix = st3
            x = xb[pl.ds(vi * 16, 16)]
            nl = nb[pl.ds(vi * 16, 16)]
            k = _keys_of(x)
            idxv = base + ci * CHUNK + vi * 16 + lane
            tiem = k == kstar
            offs = jnp.minimum(jnp.max(cnt), TIE_CAP)
            plsc.store_compressed(tieidx.at[pl.ds(offs, 16)], idxv,
                                  mask=tiem)
            tnl = jnp.where(tiem, nl, jnp.inf)
            plsc.store_compressed(tienlx.at[pl.ds(offs, 16)], tnl,
                                  mask=tiem)
            cnt = cnt + plsc.all_reduce_population_count(tiem)
            tbet = tnl < manl
            manl = jnp.where(tbet, tnl, manl)
            maix = jnp.where(tbet, idxv, maix)
            return cnt, manl, maix
        return lax.fori_loop(0, NVEC, sb, (cnt, manl, maix))

    def sel_compute(xb, nb, ci, st):
        best, bidx, cnt, manl, maix = st
        fzero = jnp.zeros((16,), jnp.int32)

        def vec_body(vi, st2):
            best, bidx, tacc = st2
            for u in range(UNROLL):
                x = xb[pl.ds(vi * (16 * UNROLL) + u * 16, 16)]
                nl = nb[pl.ds(vi * (16 * UNROLL) + u * 16, 16)]
                w = jnp.exp(x)
                k = _keys_of(x)
                idxv = (base + ci * CHUNK + vi * (16 * UNROLL) + u * 16
                        + lane)
                certain = k > kstar
                sc = jnp.where(certain, nl / w, jnp.inf)
                better = sc < best
                best = jnp.where(better, sc, best)
                bidx = jnp.where(better, idxv, bidx)
                tacc = tacc | jnp.where(k == kstar, 1, 0)
            return best, bidx, tacc

        best, bidx, tacc = lax.fori_loop(0, NVEC // UNROLL, vec_body,
                                         (best, bidx, fzero))
        any_tie = jnp.max(tacc) > 0
        cnt, manl, maix = lax.cond(
            any_tie,
            lambda: slow_ties(xb, nb, ci, cnt, manl, maix),
            lambda: (cnt, manl, maix))
        return best, bidx, cnt, manl, maix

    def xsrc(ci):
        return logits_hbm.at[pl.ds(rbase + ci * CHUNK, CHUNK)]

    def nsrc(ci):
        return nlx_hbm.at[pl.ds(rbase + ci * CHUNK, CHUNK)]

    pltpu.async_copy(xsrc(0), xbuf, sem0)
    pltpu.async_copy(nsrc(0), nbuf, sem2)

    def sel_pair(p, st):
        pltpu.async_copy(xsrc(2 * p + 1), xbuf2, sem1)
        pltpu.async_copy(nsrc(2 * p + 1), nbuf2, sem3)
        pltpu.make_async_copy(xsrc(2 * p), xbuf, sem0).wait()
        pltpu.make_async_copy(nsrc(2 * p), nbuf, sem2).wait()
        st = sel_compute(xbuf, nbuf, 2 * p, st)

        @pl.when(p < NPAIR - 1)
        def _pref():
            pltpu.async_copy(xsrc(2 * p + 2), xbuf, sem0)
            pltpu.async_copy(nsrc(2 * p + 2), nbuf, sem2)
        pltpu.make_async_copy(xsrc(2 * p + 1), xbuf2, sem1).wait()
        pltpu.make_async_copy(nsrc(2 * p + 1), nbuf2, sem3).wait()
        st = sel_compute(xbuf2, nbuf2, 2 * p + 1, st)
        return st

    izero = jnp.zeros((16,), jnp.int32)
    best, bidx, cnt, manl, maix = lax.fori_loop(
        0, NPAIR, sel_pair, (inf16, izero, izero, inf16, izero))

    bigi = jnp.int32(2 ** 30)
    bestm = jnp.min(best)
    besti = jnp.min(jnp.where(best == bestm, bidx, bigi))
    ncnt = jnp.max(cnt)
    manl_s = jnp.min(manl)
    maix_s = jnp.min(jnp.where(manl == manl_s, maix, bigi))

    recf = jnp.where(lane == 0, bestm, jnp.where(lane == 1, manl_s, 0.0))
    reci = jnp.where(lane == 0, besti,
                     jnp.where(lane == 1, ncnt,
                               jnp.where(lane == 2, maix_s, 0)))
    recfv[...] = recf
    reciv[...] = reci
    myslot = row_l * 2 + half
    pltpu.sync_copy(recfv, recf_sh.at[pl.ds(myslot * REC_STRIDE, 16)])
    pltpu.sync_copy(reciv, reci_sh.at[pl.ds(myslot * REC_STRIDE, 16)])
    pltpu.sync_copy(tieidx, tidx_sh.at[pl.ds(myslot * REC_STRIDE, TIE_WORDS)])
    pltpu.sync_copy(tienlx, tnlx_sh.at[pl.ds(myslot * REC_STRIDE, TIE_WORDS)])
    plsc.subcore_barrier()

    s0 = row_l * 2
    pltpu.sync_copy(recf_sh.at[pl.ds(s0 * REC_STRIDE, 16)], recfv)
    pltpu.sync_copy(reci_sh.at[pl.ds(s0 * REC_STRIDE, 16)], reciv)
    rf0 = recfv[...]
    ri0 = reciv[...]
    bm0 = exf(rf0, 0)
    manl0 = exf(rf0, 1)
    bi0 = exi(ri0, 0)
    n0 = exi(ri0, 1)
    mai0 = exi(ri0, 2)
    s1 = row_l * 2 + 1
    pltpu.sync_copy(recf_sh.at[pl.ds(s1 * REC_STRIDE, 16)], recfv)
    pltpu.sync_copy(reci_sh.at[pl.ds(s1 * REC_STRIDE, 16)], reciv)
    rf1 = recfv[...]
    ri1 = reciv[...]
    bm1 = exf(rf1, 0)
    manl1 = exf(rf1, 1)
    bi1 = exi(ri1, 0)
    n1 = exi(ri1, 1)
    mai1 = exi(ri1, 2)

    k0 = jnp.minimum(r, n0)
    k1 = jnp.minimum(jnp.maximum(r - n0, 0), n1)
    use_all = jnp.logical_or(r >= n0 + n1,
                             jnp.logical_or(k0 > TIE_CAP, k1 > TIE_CAP))

    def walk(hh, kk, st):
        hs = (row_l * 2 + hh) * REC_STRIDE
        pltpu.sync_copy(tidx_sh.at[pl.ds(hs, TIE_WORDS)], tieidx)
        pltpu.sync_copy(tnlx_sh.at[pl.ds(hs, TIE_WORDS)], tienlx)

        def wb(j, st2):
            mn, mi = st2
            lm = (j * 16 + lane) < kk
            vals = jnp.where(lm, tienlx[pl.ds(j * 16, 16)], jnp.inf)
            ids = tieidx[pl.ds(j * 16, 16)]
            bet = vals < mn
            return jnp.where(bet, vals, mn), jnp.where(bet, ids, mi)
        return lax.fori_loop(0, TIE_CAP // 16, wb, st)

    tmn, tmi = walk(0, k0, (inf16, izero))
    tmn, tmi = walk(1, k1, (tmn, tmi))
    t_nl = jnp.min(tmn)
    t_ix = jnp.min(jnp.where(tmn == t_nl, tmi, bigi))

    all_nl = jnp.where(manl0 <= manl1, manl0, manl1)
    all_ix = jnp.where(manl0 <= manl1, mai0, mai1)
    tie_nl = jnp.where(use_all, all_nl, t_nl)
    tie_ix = jnp.where(use_all, all_ix, t_ix)
    tie_sc = jnp.max(jnp.broadcast_to(tie_nl, (16,)) / wv_vec)

    ws = bm0
    wi = bi0
    upd = bm1 < ws
    ws = jnp.where(upd, bm1, ws)
    wi = jnp.where(upd, bi1, wi)
    updt = tie_sc < ws
    wi = jnp.where(updt, tie_ix, wi)

    fill16 = jnp.full((16,), FILL, dtype=jnp.float32)

    def fb(i, carry):
        for u in range(8):
            hist[pl.ds(i * 128 + u * 16, 16)] = fill16
        return carry
    lax.fori_loop(0, HWORDS // 128, fb, 0)
    rem = HALF - 7 * HWORDS
    for i in range(7):
        pltpu.async_copy(hist,
                         out_hbm.at[pl.ds(rbase + i * HWORDS, HWORDS)],
                         sem0)
    pltpu.async_copy(hist.at[pl.ds(0, rem)],
                     out_hbm.at[pl.ds(rbase + 7 * HWORDS, rem)], sem0)
    for i in range(7):
        pltpu.make_async_copy(
            hist, out_hbm.at[pl.ds(rbase + i * HWORDS, HWORDS)],
            sem0).wait()
    pltpu.make_async_copy(
        hist.at[pl.ds(0, rem)],
        out_hbm.at[pl.ds(rbase + 7 * HWORDS, rem)], sem0).wait()

    own = jnp.logical_and(wi >= base, wi < base + HALF)

    @pl.when(own)
    def _patch():
        pb = wi & jnp.int32(-16)
        off = wi - pb
        patch[...] = jnp.where(lane == off, MARK, FILL)
        dst = pl.multiple_of(row * VOCAB + pb, 16)
        pltpu.sync_copy(patch, out_hbm.at[pl.ds(dst, 16)])


_mesh = plsc.VectorSubcoreMesh(core_axis_name="c", subcore_axis_name="s",
                               num_cores=2, num_subcores=16)

_sc_call = functools.partial(
    pl.kernel,
    out_type=jax.ShapeDtypeStruct((B * VOCAB,), jnp.float32),
    mesh=_mesh,
    compiler_params=pltpu.CompilerParams(needs_layout_passes=False),
    scratch_types=[
        pltpu.VMEM((CHUNK,), jnp.float32),
        pltpu.VMEM((CHUNK,), jnp.float32),
        pltpu.VMEM((CHUNK,), jnp.float32),
        pltpu.VMEM((CHUNK,), jnp.float32),
        pltpu.VMEM((HWORDS,), jnp.float32),
        pltpu.VMEM((TIE_WORDS,), jnp.int32),
        pltpu.VMEM((TIE_WORDS,), jnp.float32),
        pltpu.VMEM((16,), jnp.float32),
        pltpu.VMEM((16,), jnp.int32),
        pltpu.VMEM((16,), jnp.float32),
        pltpu.SemaphoreType.DMA,
        pltpu.SemaphoreType.DMA,
        pltpu.SemaphoreType.DMA,
        pltpu.SemaphoreType.DMA,
        pltpu.VMEM_SHARED((ROWS_PER_SC * 2 * MERGE_STRIDE,), jnp.float32),
        pltpu.VMEM_SHARED((ROWS_PER_SC * 2 * REC_STRIDE,), jnp.float32),
        pltpu.VMEM_SHARED((ROWS_PER_SC * 2 * REC_STRIDE,), jnp.int32),
        pltpu.VMEM_SHARED((ROWS_PER_SC * 2 * REC_STRIDE,), jnp.int32),
        pltpu.VMEM_SHARED((ROWS_PER_SC * 2 * REC_STRIDE,), jnp.float32),
    ],
)(_sc_body)


def kernel(input_ids, logits, input_vector, random_vectors):
    del input_ids
    key0 = jax.random.key(SEED)
    powers = 2 ** jnp.arange(BBITS, dtype=jnp.int32)

    def row_hash(vec):
        proj = random_vectors @ vec
        bits = (proj > 0).astype(jnp.int32)
        return jnp.sum(bits * powers)

    hashes = jax.vmap(row_hash)(input_vector)
    keys = jax.vmap(lambda h: jax.random.fold_in(key0, h))(hashes)
    xi = jax.vmap(lambda k: jax.random.uniform(
        k, (VOCAB,), dtype=jnp.float32, minval=1e-9, maxval=1.0))(keys)
    nlx = -jnp.log(xi)
    flat = _sc_call(logits.reshape(-1), nlx.reshape(-1))
    return flat.reshape(B, VOCAB)
